# baseline (speedup 1.0000x reference)
import functools

import jax
import jax.numpy as jnp
from jax import lax
from jax.experimental import pallas as pl
from jax.experimental.pallas import tpu as pltpu
from jax.experimental.pallas import tpu_sc as plsc

_CHUNK = 64
_W = 128
_L = 16
_R = 4


def _sc_body(x_hbm, m_hbm, o_hbm, *bufs):
    xbufs, mbufs, obufs = bufs[0:_R], bufs[_R:2 * _R], bufs[2 * _R:3 * _R]
    sxs = bufs[3 * _R:4 * _R]
    sms = bufs[4 * _R:5 * _R]
    sos = bufs[5 * _R:6 * _R]

    b_, n, d = x_hbm.shape
    ncores = 2
    nsub = 16
    nw = ncores * nsub
    nslabs = d // _W
    ntasks = b_ * nslabs
    nchunks = n // _CHUNK
    wid = lax.axis_index("s") * ncores + lax.axis_index("c")

    def sl(ref, b, col, ch):
        return ref.at[b, pl.ds(ch * _CHUNK, _CHUNK), pl.ds(col, _W)]

    def make_step(r):
        def step(i, accs):
            out = []
            for s in range(_W // _L):
                a = accs[s] + (xbufs[r][i, s * _L:(s + 1) * _L]
                               * mbufs[r][i, s * _L:(s + 1) * _L])
                obufs[r][i, s * _L:(s + 1) * _L] = a
                out.append(a)
            return tuple(out)
        return step

    for t in range(ntasks // nw):
        g = wid + nw * t
        b = g // nslabs
        col = (g % nslabs) * _W

        for r in range(_R - 1):
            pltpu.async_copy(sl(x_hbm, b, col, r), xbufs[r], sxs[r])
            pltpu.async_copy(sl(m_hbm, b, col, r), mbufs[r], sms[r])

        def group_body(k, accs):
            for r in range(_R):
                ch = _R * k + r
                pre = ch + _R - 1
                rp = (r + _R - 1) % _R

                @pl.when(pre < nchunks)
                def _():
                    pltpu.async_copy(sl(x_hbm, b, col, pre), xbufs[rp], sxs[rp])
                    pltpu.async_copy(sl(m_hbm, b, col, pre), mbufs[rp], sms[rp])

                pltpu.make_async_copy(
                    sl(x_hbm, b, col, 0), xbufs[r], sxs[r]).wait()
                pltpu.make_async_copy(
                    sl(m_hbm, b, col, 0), mbufs[r], sms[r]).wait()

                @pl.when(ch >= _R)
                def _():
                    pltpu.make_async_copy(
                        obufs[r], sl(o_hbm, b, col, 0), sos[r]).wait()

                accs = lax.fori_loop(0, _CHUNK, make_step(r), accs)
                pltpu.async_copy(obufs[r], sl(o_hbm, b, col, ch), sos[r])
            return accs

        accs = tuple(jnp.zeros((_L,), jnp.float32) for _ in range(_W // _L))
        lax.fori_loop(0, nchunks // _R, group_body, accs)

        for r in range(_R):
            pltpu.make_async_copy(
                obufs[r], sl(o_hbm, b, col, 0), sos[r]).wait()


def kernel(x, mask):
    b, n, d = x.shape
    mesh = plsc.VectorSubcoreMesh(core_axis_name="c", subcore_axis_name="s")
    buf = pltpu.VMEM((_CHUNK, _W), jnp.float32)
    sc_call = functools.partial(
        pl.kernel,
        mesh=mesh,
        out_type=jax.ShapeDtypeStruct((b, n, d), jnp.float32),
        scratch_types=(
            [buf] * (3 * _R) + [pltpu.SemaphoreType.DMA] * (3 * _R)
        ),
    )(_sc_body)
    return sc_call(x, mask)


# speedup vs baseline: 1.0022x; 1.0022x over previous
import functools

import jax
import jax.numpy as jnp
from jax import lax
from jax.experimental import pallas as pl
from jax.experimental.pallas import tpu as pltpu
from jax.experimental.pallas import tpu_sc as plsc

_CHUNK = 64
_W = 128
_L = 16
_R = 4


def _sc_body(x_hbm, m_hbm, o_hbm, *bufs):
    xbufs, mbufs, obufs = bufs[0:_R], bufs[_R:2 * _R], bufs[2 * _R:3 * _R]
    sxs = bufs[3 * _R:4 * _R]
    sms = bufs[4 * _R:5 * _R]
    sos = bufs[5 * _R:6 * _R]

    b_, n, d = x_hbm.shape
    ncores = 2
    nsub = 16
    nw = ncores * nsub
    nslabs = d // _W
    ntasks_per_w = (b_ * nslabs) // nw
    nchunks = n // _CHUNK
    nflat = ntasks_per_w * nchunks
    wid = lax.axis_index("s") * ncores + lax.axis_index("c")

    def chunk_slice(ref, ci):
        t = ci // nchunks
        local = ci % nchunks
        g = wid + nw * t
        b = g // nslabs
        col = (g % nslabs) * _W
        return ref.at[b, pl.ds(local * _CHUNK, _CHUNK), pl.ds(col, _W)]

    def make_step(r):
        def step(i2, accs):
            out = accs
            for u in range(2):
                i = i2 * 2 + u
                nxt = []
                for s in range(_W // _L):
                    a = out[s] + (xbufs[r][i, s * _L:(s + 1) * _L]
                                  * mbufs[r][i, s * _L:(s + 1) * _L])
                    obufs[r][i, s * _L:(s + 1) * _L] = a
                    nxt.append(a)
                out = tuple(nxt)
            return out
        return step

    for r in range(_R - 1):
        pltpu.async_copy(chunk_slice(x_hbm, r), xbufs[r], sxs[r])
        pltpu.async_copy(chunk_slice(m_hbm, r), mbufs[r], sms[r])

    def group_body(k, accs):
        for r in range(_R):
            ci = _R * k + r
            pre = ci + _R - 1
            rp = (r + _R - 1) % _R

            @pl.when(pre < nflat)
            def _():
                pltpu.async_copy(chunk_slice(x_hbm, pre), xbufs[rp], sxs[rp])
                pltpu.async_copy(chunk_slice(m_hbm, pre), mbufs[rp], sms[rp])

            pltpu.make_async_copy(
                chunk_slice(x_hbm, 0), xbufs[r], sxs[r]).wait()
            pltpu.make_async_copy(
                chunk_slice(m_hbm, 0), mbufs[r], sms[r]).wait()

            @pl.when(ci >= _R)
            def _():
                pltpu.make_async_copy(
                    obufs[r], chunk_slice(o_hbm, 0), sos[r]).wait()

            fresh = (ci % nchunks) == 0
            accs = tuple(
                jnp.where(fresh, jnp.zeros((_L,), jnp.float32), a)
                for a in accs)
            accs = lax.fori_loop(0, _CHUNK // 2, make_step(r), accs)
            pltpu.async_copy(obufs[r], chunk_slice(o_hbm, ci), sos[r])
        return accs

    accs = tuple(jnp.zeros((_L,), jnp.float32) for _ in range(_W // _L))
    lax.fori_loop(0, nflat // _R, group_body, accs)

    for r in range(_R):
        pltpu.make_async_copy(obufs[r], chunk_slice(o_hbm, 0), sos[r]).wait()


def kernel(x, mask):
    b, n, d = x.shape
    mesh = plsc.VectorSubcoreMesh(core_axis_name="c", subcore_axis_name="s")
    buf = pltpu.VMEM((_CHUNK, _W), jnp.float32)
    sc_call = functools.partial(
        pl.kernel,
        mesh=mesh,
        out_type=jax.ShapeDtypeStruct((b, n, d), jnp.float32),
        scratch_types=(
            [buf] * (3 * _R) + [pltpu.SemaphoreType.DMA] * (3 * _R)
        ),
    )(_sc_body)
    return sc_call(x, mask)
